# Initial kernel scaffold; baseline (speedup 1.0000x reference)
import jax
import jax.numpy as jnp
from jax.experimental import pallas as pl


def kernel(x, W1, W2):
    raise NotImplementedError("write your pallas kernel here")


# speedup vs baseline: 3.5707x; 3.5707x over previous
import jax
import jax.numpy as jnp
from jax.experimental import pallas as pl

D_IN = 2048
D_OUT = 2048
DEPTH = 4
N_NODES = 2 ** (DEPTH + 1) - 1
N_PAD = 32


def _fff_block(x_ref, w1_ref, w2_ref, o_ref):
    x = x_ref[...]
    w1 = w1_ref[...]
    w2 = w2_ref[...]
    logits = jax.lax.dot_general(
        x, w1, (((1,), (1,)), ((), ())),
        preferred_element_type=jnp.float32,
        precision=jax.lax.Precision.HIGHEST)
    tb = x.shape[0]
    iota = jax.lax.broadcasted_iota(jnp.int32, (tb, N_PAD), 1)
    node = jnp.zeros((tb, 1), jnp.int32)
    coef = jnp.zeros((tb, N_PAD), jnp.float32)
    for _ in range(DEPTH + 1):
        onehot = iota == node
        l = jnp.sum(jnp.where(onehot, logits, 0.0), axis=1, keepdims=True)
        coef = jnp.where(onehot, jnp.maximum(l, 0.0), coef)
        node = 2 * node + 1 + (l > 0).astype(jnp.int32)
    o_ref[...] = jax.lax.dot_general(
        coef, w2, (((1,), (0,)), ((), ())),
        preferred_element_type=jnp.float32,
        precision=jax.lax.Precision.HIGHEST)


def kernel(x, W1, W2):
    b = x.shape[0] * x.shape[1]
    xf = x.reshape(b, D_IN)
    w1p = jnp.pad(W1, ((0, N_PAD - N_NODES), (0, 0)))
    w2p = jnp.pad(W2, ((0, N_PAD - N_NODES), (0, 0)))
    tb = 1024
    return pl.pallas_call(
        _fff_block,
        grid=(b // tb,),
        in_specs=[
            pl.BlockSpec((tb, D_IN), lambda i: (i, 0)),
            pl.BlockSpec((N_PAD, D_IN), lambda i: (0, 0)),
            pl.BlockSpec((N_PAD, D_OUT), lambda i: (0, 0)),
        ],
        out_specs=pl.BlockSpec((tb, D_OUT), lambda i: (i, 0)),
        out_shape=jax.ShapeDtypeStruct((b, D_OUT), jnp.float32),
    )(xf, w1p, w2p)


# speedup vs baseline: 8.6954x; 2.4352x over previous
import jax
import jax.numpy as jnp
from jax.experimental import pallas as pl

D_IN = 2048
D_OUT = 2048
DEPTH = 4
N_NODES = 2 ** (DEPTH + 1) - 1
N_PAD = 32


def _fff_block(x_ref, w1_ref, w2_ref, o_ref):
    x = x_ref[...]
    w1 = w1_ref[...]
    w2 = w2_ref[...]
    logits = jax.lax.dot_general(
        x, w1, (((1,), (1,)), ((), ())),
        preferred_element_type=jnp.float32,
        precision=jax.lax.Precision.HIGHEST)
    tb = x.shape[0]
    iota = jax.lax.broadcasted_iota(jnp.int32, (tb, N_PAD), 1)
    node = jnp.zeros((tb, 1), jnp.int32)
    coef = jnp.zeros((tb, N_PAD), jnp.float32)
    for _ in range(DEPTH + 1):
        onehot = iota == node
        l = jnp.sum(jnp.where(onehot, logits, 0.0), axis=1, keepdims=True)
        coef = jnp.where(onehot, jnp.maximum(l, 0.0), coef)
        node = 2 * node + 1 + (l > 0).astype(jnp.int32)
    o_ref[...] = jax.lax.dot_general(
        coef.astype(jnp.bfloat16), w2, (((1,), (0,)), ((), ())),
        preferred_element_type=jnp.float32)


def kernel(x, W1, W2):
    b = x.shape[0] * x.shape[1]
    xf = x.reshape(b, D_IN)
    w1p = jnp.pad(W1, ((0, N_PAD - N_NODES), (0, 0)))
    w2p = jnp.pad(W2, ((0, N_PAD - N_NODES), (0, 0))).astype(jnp.bfloat16)
    tb = 1024
    return pl.pallas_call(
        _fff_block,
        grid=(b // tb,),
        in_specs=[
            pl.BlockSpec((tb, D_IN), lambda i: (i, 0)),
            pl.BlockSpec((N_PAD, D_IN), lambda i: (0, 0)),
            pl.BlockSpec((N_PAD, D_OUT), lambda i: (0, 0)),
        ],
        out_specs=pl.BlockSpec((tb, D_OUT), lambda i: (i, 0)),
        out_shape=jax.ShapeDtypeStruct((b, D_OUT), jnp.float32),
    )(xf, w1p, w2p)


# speedup vs baseline: 8.6989x; 1.0004x over previous
import jax
import jax.numpy as jnp
from jax.experimental import pallas as pl

D_IN = 2048
D_OUT = 2048
DEPTH = 4
N_NODES = 2 ** (DEPTH + 1) - 1
N_PAD = 32
M_CHUNKS = 1


def _dot_nt(a, b):
    return jax.lax.dot_general(a, b, (((1,), (1,)), ((), ())),
                               preferred_element_type=jnp.float32)


def _fff_block(x_ref, w1_ref, w2_ref, o_ref):
    w1p = w1_ref[...]
    w1h = w1p.astype(jnp.bfloat16)
    rw = w1p - w1h.astype(jnp.float32)
    w1l = rw.astype(jnp.bfloat16)
    w1l2 = (rw - w1l.astype(jnp.float32)).astype(jnp.bfloat16)
    w2 = w2_ref[...]
    tb = x_ref.shape[0]
    mc = tb // M_CHUNKS
    for c in range(M_CHUNKS):
        x = x_ref[pl.ds(c * mc, mc), :]
        xh = x.astype(jnp.bfloat16)
        r1 = x - xh.astype(jnp.float32)
        xl = r1.astype(jnp.bfloat16)
        xl2 = (r1 - xl.astype(jnp.float32)).astype(jnp.bfloat16)
        logits = (_dot_nt(xh, w1h)
                  + (_dot_nt(xh, w1l) + _dot_nt(xl, w1h))
                  + (_dot_nt(xl, w1l) + _dot_nt(xl2, w1h) + _dot_nt(xh, w1l2)))
        iota = jax.lax.broadcasted_iota(jnp.int32, (mc, N_PAD), 1)
        node = jnp.zeros((mc, 1), jnp.int32)
        coef = jnp.zeros((mc, N_PAD), jnp.float32)
        for _ in range(DEPTH + 1):
            onehot = iota == node
            l = jnp.sum(jnp.where(onehot, logits, 0.0), axis=1, keepdims=True)
            coef = jnp.where(onehot, jnp.maximum(l, 0.0), coef)
            node = 2 * node + 1 + (l > 0).astype(jnp.int32)
        o_ref[pl.ds(c * mc, mc), :] = jax.lax.dot_general(
            coef.astype(jnp.bfloat16), w2, (((1,), (0,)), ((), ())),
            preferred_element_type=jnp.float32)


def kernel(x, W1, W2):
    b = x.shape[0] * x.shape[1]
    xf = x.reshape(b, D_IN)
    w1p = jnp.pad(W1, ((0, N_PAD - N_NODES), (0, 0)))
    w2p = jnp.pad(W2, ((0, N_PAD - N_NODES), (0, 0))).astype(jnp.bfloat16)
    tb = 1024
    tbl_spec = pl.BlockSpec((N_PAD, D_IN), lambda i: (0, 0))
    return pl.pallas_call(
        _fff_block,
        grid=(b // tb,),
        in_specs=[
            pl.BlockSpec((tb, D_IN), lambda i: (i, 0)),
            tbl_spec,
            pl.BlockSpec((N_PAD, D_OUT), lambda i: (0, 0)),
        ],
        out_specs=pl.BlockSpec((tb, D_OUT), lambda i: (i, 0)),
        out_shape=jax.ShapeDtypeStruct((b, D_OUT), jnp.float32),
    )(xf, w1p, w2p)
